# Initial kernel scaffold; baseline (speedup 1.0000x reference)
import jax
import jax.numpy as jnp
from jax.experimental import pallas as pl


def kernel(unary_tensor, binary_tensor, W_embed, b_embed, W_root, W_rel, b_gnn, W_d, b_d, W_b, b_b):
    raise NotImplementedError("write your pallas kernel here")


# speedup vs baseline: 1635.3585x; 1635.3585x over previous
import jax
import jax.numpy as jnp
from jax.experimental import pallas as pl
from jax.experimental.pallas import tpu as pltpu

_T, _B, _OBJ, _FEAT, _R, _EMB, _NBL, _MP = 2, 8, 128, 64, 3, 16, 2, 2
_G = _T * _B


def _gnn_body(adj_ref, unary_ref, We_ref, be_ref, Wr_ref, Wrel_ref, bg_ref,
              Wd_ref, bd_ref, Wb_ref, bb_ref, out_ref):
    f32 = jnp.float32
    adj = adj_ref[0]

    an_blocks = []
    for r in range(_R):
        a = (adj[r] != 0).astype(f32)
        deg = jnp.sum(a, axis=0, keepdims=True)
        norm = 1.0 / jnp.maximum(deg, 1.0)
        an_blocks.append(a * norm)
    an = jnp.concatenate(an_blocks, axis=0)

    x = jnp.dot(unary_ref[0], We_ref[...],
                preferred_element_type=f32) + be_ref[...]

    for l in range(_NBL):
        w_root = Wr_ref[l]
        b = bg_ref[l:l + 1, :]
        for _ in range(_MP):
            t = jnp.concatenate(
                [jnp.dot(x, Wrel_ref[l, r], preferred_element_type=f32)
                 for r in range(_R)], axis=0)
            agg = jax.lax.dot_general(
                an, t, (((0,), (0,)), ((), ())),
                preferred_element_type=f32)
            root = jnp.dot(x, w_root, preferred_element_type=f32)
            x = jnp.maximum(agg + root + b, 0.0)

    pooled = jnp.max(x, axis=0, keepdims=True)
    h = jnp.maximum(jnp.dot(pooled, Wd_ref[...],
                            preferred_element_type=f32) + bd_ref[...], 0.0)
    val = jnp.sum(h * Wb_ref[...], axis=1, keepdims=True) + bb_ref[...]
    out_ref[...] = jnp.broadcast_to(val, (1, 1, 128))


def kernel(unary_tensor, binary_tensor, W_embed, b_embed, W_root, W_rel,
           b_gnn, W_d, b_d, W_b, b_b):
    adj = binary_tensor.reshape(_G, _OBJ, _OBJ, _R).transpose(0, 3, 1, 2)
    unary = unary_tensor.astype(jnp.float32).reshape(_G, _OBJ, _FEAT)
    be = b_embed.reshape(1, _EMB)
    bd = b_d.reshape(1, 128)
    wb = W_b.reshape(1, 128)
    bb = b_b.reshape(1, 1)

    full = lambda *shape: pl.BlockSpec(shape, lambda g: (0,) * len(shape))
    out = pl.pallas_call(
        _gnn_body,
        grid=(_G,),
        in_specs=[
            pl.BlockSpec((1, _R, _OBJ, _OBJ), lambda g: (g, 0, 0, 0)),
            pl.BlockSpec((1, _OBJ, _FEAT), lambda g: (g, 0, 0)),
            full(_FEAT, _EMB),
            full(1, _EMB),
            full(_NBL, _EMB, _EMB),
            full(_NBL, _R, _EMB, _EMB),
            full(_NBL, _EMB),
            full(_EMB, 128),
            full(1, 128),
            full(1, 128),
            full(1, 1),
        ],
        out_specs=pl.BlockSpec((1, 1, 128), lambda g: (g, 0, 0)),
        out_shape=jax.ShapeDtypeStruct((_G, 1, 128), jnp.float32),
        compiler_params=pltpu.CompilerParams(
            dimension_semantics=("parallel",)),
    )(adj, unary, W_embed, be, W_root, W_rel, b_gnn, W_d, bd, wb, bb)
    return out[:, 0, 0].reshape(_T, _B)
